# baseline (speedup 1.0000x reference)
import functools

import numpy as np
import jax
import jax.numpy as jnp
from jax import lax
from jax.experimental import pallas as pl
from jax.experimental.pallas import tpu as pltpu
from jax.experimental.pallas import tpu_sc as plsc

_NC, _NS, _L = 2, 16, 16


def kernel(lengths, table):
    n = lengths.shape[0]
    rows, d = table.shape
    nw = _NC * _NS
    n_per_w = n // nw
    groups = n_per_w // _L

    flat = np.arange(_L * d)
    rk_np = (flat // d).reshape(d, _L).astype(np.int32)
    ck_np = (flat % d).reshape(d, _L).astype(np.int32)

    mesh = plsc.VectorSubcoreMesh(
        core_axis_name="c", subcore_axis_name="s",
        num_cores=_NC, num_subcores=_NS)

    @functools.partial(
        pl.kernel,
        out_type=jax.ShapeDtypeStruct((n * d,), jnp.float32),
        mesh=mesh,
        compiler_params=pltpu.CompilerParams(needs_layout_passes=False),
        scratch_types=[
            pltpu.VMEM((n_per_w,), jnp.int32),
            pltpu.VMEM((rows * d,), jnp.float32),
            pltpu.VMEM((n_per_w * d,), jnp.float32),
        ],
    )
    def run(lengths_hbm, table_hbm, out_hbm, len_v, tab_v, out_v):
        wid = lax.axis_index("s") * _NC + lax.axis_index("c")
        base = wid * n_per_w
        pltpu.sync_copy(lengths_hbm.at[pl.ds(base, n_per_w)], len_v)
        pltpu.sync_copy(table_hbm, tab_v)

        lane = lax.iota(jnp.int32, _L)
        rk_c, fpos_c = [], []
        for k in range(d):
            p = lane + (_L * k)
            rk = (p * 13108) >> 18
            rk_c.append(rk)
            fpos_c.append(p - rk * d)

        @plsc.parallel_loop(0, groups, 1, unroll=2)
        def body(g):
            lv = len_v[pl.ds(g * _L, _L)]
            f = lv.astype(jnp.float32)
            e2 = (lax.bitcast_convert_type(f, jnp.int32) >> 23) - 125
            idx = jnp.where(lv < 4, lv, e2)
            tpos = idx * d
            gbase = g * (_L * d)
            for k in range(d):
                fpos = tpos.at[rk_c[k]].get(mode="promise_in_bounds") + fpos_c[k]
                vals = plsc.load_gather(tab_v, [fpos])
                out_v[pl.ds(gbase + k * _L, _L)] = vals

        pltpu.sync_copy(out_v, out_hbm.at[pl.ds(base * d, n_per_w * d)])

    return run(lengths, table.reshape(-1)).reshape(n, d)


# speedup vs baseline: 1.0045x; 1.0045x over previous
import functools

import numpy as np
import jax
import jax.numpy as jnp
from jax import lax
from jax.experimental import pallas as pl
from jax.experimental.pallas import tpu as pltpu
from jax.experimental.pallas import tpu_sc as plsc

_NC, _NS, _L = 2, 16, 16


def kernel(lengths, table):
    n = lengths.shape[0]
    rows, d = table.shape
    nw = _NC * _NS
    n_per_w = n // nw
    groups = n_per_w // _L

    flat = np.arange(_L * d)
    rk_np = (flat // d).reshape(d, _L).astype(np.int32)
    ck_np = (flat % d).reshape(d, _L).astype(np.int32)

    mesh = plsc.VectorSubcoreMesh(
        core_axis_name="c", subcore_axis_name="s",
        num_cores=_NC, num_subcores=_NS)

    @functools.partial(
        pl.kernel,
        out_type=jax.ShapeDtypeStruct((n * d,), jnp.float32),
        mesh=mesh,
        compiler_params=pltpu.CompilerParams(needs_layout_passes=False),
        scratch_types=[
            pltpu.VMEM((n_per_w,), jnp.int32),
            pltpu.VMEM((rows * d,), jnp.float32),
            pltpu.VMEM((n_per_w * d,), jnp.float32),
            pltpu.SemaphoreType.DMA,
            pltpu.SemaphoreType.DMA,
        ],
    )
    def run(lengths_hbm, table_hbm, out_hbm, len_v, tab_v, out_v,
            sem_in, sem_out):
        wid = lax.axis_index("s") * _NC + lax.axis_index("c")
        base = wid * n_per_w
        cp_len = pltpu.async_copy(
            lengths_hbm.at[pl.ds(base, n_per_w)], len_v, sem_in)
        cp_tab = pltpu.async_copy(table_hbm, tab_v, sem_in)
        cp_len.wait()
        cp_tab.wait()

        lane = lax.iota(jnp.int32, _L)
        rk_c, fpos_c = [], []
        for k in range(d):
            p = lane + (_L * k)
            rk = (p * 13108) >> 18
            rk_c.append(rk)
            fpos_c.append(p - rk * d)

        half = groups // 2
        hwords = half * _L * d

        def body(g):
            lv = len_v[pl.ds(g * _L, _L)]
            f = lv.astype(jnp.float32)
            e2 = (lax.bitcast_convert_type(f, jnp.int32) >> 23) - 125
            idx = jnp.where(lv < 4, lv, e2)
            tpos = idx * d
            gbase = g * (_L * d)
            for k in range(d):
                fpos = tpos.at[rk_c[k]].get(mode="promise_in_bounds") + fpos_c[k]
                vals = plsc.load_gather(tab_v, [fpos])
                out_v[pl.ds(gbase + k * _L, _L)] = vals

        plsc.parallel_loop(0, half, 1, unroll=1)(body)
        cp_half = pltpu.async_copy(
            out_v.at[pl.ds(0, hwords)],
            out_hbm.at[pl.ds(base * d, hwords)], sem_out)
        plsc.parallel_loop(half, groups, 1, unroll=1)(body)
        cp_rest = pltpu.async_copy(
            out_v.at[pl.ds(hwords, hwords)],
            out_hbm.at[pl.ds(base * d + hwords, hwords)], sem_out)
        cp_half.wait()
        cp_rest.wait()

    return run(lengths, table.reshape(-1)).reshape(n, d)
